# baseline (speedup 1.0000x reference)
import functools

import jax
import jax.numpy as jnp
from jax import lax
from jax.experimental import pallas as pl
from jax.experimental.pallas import tpu as pltpu
from jax.experimental.pallas import tpu_sc as plsc


def _lrelu(x):
    return jnp.where(x >= 0, x, 0.2 * x)


def _sigmoid(x):
    return 1.0 / (1.0 + jnp.exp(-x))


_BN_RSQRT = 1.0 / (1.0 + 1e-5) ** 0.5
KMAX = 40


def _pd_topk_body(rows_ref, full_ref, idx_ref, avg_ref, *, n_points):
    xr = rows_ref[0]
    xf = full_ref[0]
    inner = lax.dot_general(xr, xf, (((1,), (1,)), ((), ())),
                            preferred_element_type=jnp.float32)
    xxr = jnp.sum(xr * xr, axis=1)
    xxf = jnp.sum(xf * xf, axis=1)
    pd = 2.0 * inner - xxr[:, None] - xxf[None, :]
    bn = pd.shape[0]
    colio = lax.broadcasted_iota(jnp.int32, (bn, n_points), 1)
    vals = []
    idxs = []
    for _ in range(KMAX):
        gm = jnp.max(pd, axis=1)
        pos = jnp.min(jnp.where(pd == gm[:, None], colio, n_points), axis=1)
        vals.append(gm)
        idxs.append(pos)
        pd = jnp.where(colio == pos[:, None], -jnp.inf, pd)
    idx_ref[0] = jnp.stack(idxs, axis=-1)
    avg_ref[0, 0] = sum(vals[1:11]) * 0.1


def _pd_topk(xt):
    B, N, C = xt.shape
    BN = 256
    return pl.pallas_call(
        functools.partial(_pd_topk_body, n_points=N),
        grid=(B, N // BN),
        in_specs=[
            pl.BlockSpec((1, BN, C), lambda b, n: (b, n, 0)),
            pl.BlockSpec((1, N, C), lambda b, n: (b, 0, 0)),
        ],
        out_specs=[
            pl.BlockSpec((1, BN, KMAX), lambda b, n: (b, n, 0)),
            pl.BlockSpec((1, 1, BN), lambda b, n: (b, 0, n)),
        ],
        out_shape=[
            jax.ShapeDtypeStruct((B, N, KMAX), jnp.int32),
            jax.ShapeDtypeStruct((B, 1, N), jnp.float32),
        ],
    )(xt, xt)


def _kcalc_body(avg_ref, k_ref, *, k_min, k_max):
    avg = avg_ref[...]
    mn = jnp.min(avg, axis=2, keepdims=True)
    mx = jnp.max(avg, axis=2, keepdims=True)
    nd = (avg - mn) / (mx - mn + 1e-8)
    kv = float(k_min) + float(k_max - k_min) * (1.0 - jnp.mean(nd))
    k_ref[0, 0] = jnp.clip(jnp.floor(kv).astype(jnp.int32), k_min, k_max)


def _kcalc(avg):
    return pl.pallas_call(
        functools.partial(_kcalc_body, k_min=10, k_max=KMAX),
        out_specs=pl.BlockSpec(memory_space=pltpu.SMEM),
        out_shape=jax.ShapeDtypeStruct((1, 1), jnp.int32),
    )(avg)


def _uv_body(cur_ref, wu_ref, wv_ref, bb_ref, u_ref, v_ref):
    c = cur_ref[0]
    u_ref[0] = jnp.dot(c, wu_ref[...], preferred_element_type=jnp.float32)
    v_ref[0] = (jnp.dot(c, wv_ref[...], preferred_element_type=jnp.float32)
                + bb_ref[...][0][None, :])


def _uv(cur, wu, wv, bb):
    B, N, C = cur.shape
    Co = wu.shape[1]
    return pl.pallas_call(
        _uv_body,
        grid=(B,),
        in_specs=[
            pl.BlockSpec((1, N, C), lambda b: (b, 0, 0)),
            pl.BlockSpec((C, Co), lambda b: (0, 0)),
            pl.BlockSpec((C, Co), lambda b: (0, 0)),
            pl.BlockSpec((1, Co), lambda b: (0, 0)),
        ],
        out_specs=[
            pl.BlockSpec((1, N, Co), lambda b: (b, 0, 0)),
            pl.BlockSpec((1, N, Co), lambda b: (b, 0, 0)),
        ],
        out_shape=[
            jax.ShapeDtypeStruct((B, N, Co), jnp.float32),
            jax.ShapeDtypeStruct((B, N, Co), jnp.float32),
        ],
    )(cur, wu, wv, bb)


def _sc_gather(table, idxf, n_rows_out):
    R, Co = table.shape
    mesh = plsc.VectorSubcoreMesh(core_axis_name="c", subcore_axis_name="s")
    NW = 32
    per_w = n_rows_out // NW
    ch = min(per_w, max(8, 32768 // Co))
    n_chunks = per_w // ch

    @functools.partial(
        pl.kernel, mesh=mesh,
        out_type=jax.ShapeDtypeStruct((n_rows_out, Co), jnp.float32),
        scratch_types=[
            pltpu.VMEM((ch,), jnp.int32),
            pltpu.VMEM((ch, Co), jnp.float32),
            pltpu.SemaphoreType.DMA,
        ],
    )
    def k(table_hbm, idx_hbm, out_hbm, idx_v, rows_v, sem):
        wid = lax.axis_index("s") * 2 + lax.axis_index("c")
        wbase = wid * per_w

        def body(i, carry):
            base = wbase + i * ch
            pltpu.sync_copy(idx_hbm.at[pl.ds(base, ch)], idx_v)
            pltpu.async_copy(table_hbm.at[idx_v], rows_v, sem).wait()
            pltpu.sync_copy(rows_v, out_hbm.at[pl.ds(base, ch)])
            return carry

        lax.fori_loop(0, n_chunks, body, 0)

    return k(table, idxf)


def _stats_body(e_ref, v_ref, mf_ref, sum_ref, max_ref):
    nb = pl.program_id(1)
    e = _lrelu(e_ref[0] + v_ref[0][:, None, :])
    mf = mf_ref[...][0][None, :, None]
    bsum = jnp.sum(e * mf, axis=(0, 1))[None, None, :]
    bmax = jnp.max(jnp.where(mf > 0.0, e, -jnp.inf), axis=(0, 1))[None, None, :]

    @pl.when(nb == 0)
    def _init():
        sum_ref[...] = bsum
        max_ref[...] = bmax

    @pl.when(nb > 0)
    def _acc():
        sum_ref[...] = sum_ref[...] + bsum
        max_ref[...] = jnp.maximum(max_ref[...], bmax)


def _edge_stats(E4, V, maskf):
    B, N, K, Co = E4.shape
    BN = 128
    return pl.pallas_call(
        _stats_body,
        grid=(B, N // BN),
        in_specs=[
            pl.BlockSpec((1, BN, K, Co), lambda b, n: (b, n, 0, 0)),
            pl.BlockSpec((1, BN, Co), lambda b, n: (b, n, 0)),
            pl.BlockSpec((1, K), lambda b, n: (0, 0)),
        ],
        out_specs=[
            pl.BlockSpec((1, 1, Co), lambda b, n: (b, 0, 0)),
            pl.BlockSpec((1, 1, Co), lambda b, n: (b, 0, 0)),
        ],
        out_shape=[
            jax.ShapeDtypeStruct((B, 1, Co), jnp.float32),
            jax.ShapeDtypeStruct((B, 1, Co), jnp.float32),
        ],
    )(E4, V, maskf)


def _attc_body(sum_ref, max_ref, fc1_ref, fc2_ref, kf_ref, att_ref, *, n_points):
    kf = kf_ref[0, 0]
    avg = sum_ref[:, 0, :] / (n_points * kf)
    mx = max_ref[:, 0, :]
    fc1 = fc1_ref[...]
    fc2 = fc2_ref[...]

    def fc(v):
        h = jnp.maximum(
            lax.dot_general(v, fc1, (((1,), (1,)), ((), ())),
                            preferred_element_type=jnp.float32), 0.0)
        return lax.dot_general(h, fc2, (((1,), (1,)), ((), ())),
                               preferred_element_type=jnp.float32)

    att_ref[...] = _sigmoid(fc(avg) + fc(mx))[:, None, :]


def _attc(ssum, smax, fc1, fc2, kf, n_points):
    B, _, Co = ssum.shape
    return pl.pallas_call(
        functools.partial(_attc_body, n_points=n_points),
        in_specs=[
            pl.BlockSpec((B, 1, Co), lambda: (0, 0, 0)),
            pl.BlockSpec((B, 1, Co), lambda: (0, 0, 0)),
            pl.BlockSpec(fc1.shape, lambda: (0, 0)),
            pl.BlockSpec(fc2.shape, lambda: (0, 0)),
            pl.BlockSpec(memory_space=pltpu.SMEM),
        ],
        out_specs=pl.BlockSpec((B, 1, Co), lambda: (0, 0, 0)),
        out_shape=jax.ShapeDtypeStruct((B, 1, Co), jnp.float32),
    )(ssum, smax, fc1, fc2, kf)


def _map_body(e_ref, v_ref, att_ref, mf_ref, map_ref):
    e = _lrelu(e_ref[0] + v_ref[0][:, None, :])
    x2 = e * att_ref[...][0]
    mf = mf_ref[...][0][None, :]
    avg_s = jnp.mean(x2, axis=2) * mf
    max_s = jnp.max(x2, axis=2) * mf
    map_ref[0, 0] = avg_s
    map_ref[0, 1] = max_s


def _edge_map(E4, V, attc, maskf):
    B, N, K, Co = E4.shape
    BN = 128
    return pl.pallas_call(
        _map_body,
        grid=(B, N // BN),
        in_specs=[
            pl.BlockSpec((1, BN, K, Co), lambda b, n: (b, n, 0, 0)),
            pl.BlockSpec((1, BN, Co), lambda b, n: (b, n, 0)),
            pl.BlockSpec((1, 1, Co), lambda b, n: (b, 0, 0)),
            pl.BlockSpec((1, K), lambda b, n: (0, 0)),
        ],
        out_specs=pl.BlockSpec((1, 2, BN, K), lambda b, n: (b, 0, n, 0)),
        out_shape=jax.ShapeDtypeStruct((B, 2, N, K), jnp.float32),
    )(E4, V, attc, maskf)


def _final_body(e_ref, v_ref, att_ref, mapp_ref, sw_ref, mf_ref, out_ref,
                *, bn2):
    nb = pl.program_id(1)
    e = _lrelu(e_ref[0] + v_ref[0][:, None, :])
    x2 = e * att_ref[...][0]
    acc = jnp.zeros((bn2, KMAX), jnp.float32)
    for chn in range(2):
        for di in range(5):
            for dj in range(5):
                w = sw_ref[chn * 25 + di * 5 + dj]
                sl = mapp_ref[0, chn, pl.ds(nb * bn2 + di, bn2),
                              pl.ds(dj, KMAX)]
                acc = acc + w * sl
    att_s = _sigmoid(acc)
    mf = mf_ref[...][0][None, :, None]
    val = jnp.where(mf > 0.0, x2 * att_s[:, :, None], -jnp.inf)
    out_ref[0] = jnp.max(val, axis=1)


def _edge_final(E4, V, attc, mapp, sw_flat, maskf):
    B, N, K, Co = E4.shape
    BN2 = 128
    Np4 = mapp.shape[2]
    return pl.pallas_call(
        functools.partial(_final_body, bn2=BN2),
        grid=(B, N // BN2),
        in_specs=[
            pl.BlockSpec((1, BN2, K, Co), lambda b, n: (b, n, 0, 0)),
            pl.BlockSpec((1, BN2, Co), lambda b, n: (b, n, 0)),
            pl.BlockSpec((1, 1, Co), lambda b, n: (b, 0, 0)),
            pl.BlockSpec((1, 2, Np4, K + 4), lambda b, n: (b, 0, 0, 0)),
            pl.BlockSpec(memory_space=pltpu.SMEM),
            pl.BlockSpec((1, K), lambda b, n: (0, 0)),
        ],
        out_specs=pl.BlockSpec((1, BN2, Co), lambda b, n: (b, n, 0)),
        out_shape=jax.ShapeDtypeStruct((B, N, Co), jnp.float32),
    )(E4, V, attc, mapp, sw_flat, maskf)


def _conv5_pool_body(xc_ref, w5_ref, g5_ref, b5_ref, xm_ref, xs_ref):
    nb = pl.program_id(1)
    xc = xc_ref[0]
    h = jnp.dot(xc, w5_ref[...], preferred_element_type=jnp.float32)
    s5 = (g5_ref[...][0] * _BN_RSQRT)[None, :]
    h = _lrelu(h * s5 + b5_ref[...][0][None, :])
    bmax = h.max(axis=0)[None, None, :]
    bsum = h.sum(axis=0)[None, None, :]

    @pl.when(nb == 0)
    def _init():
        xm_ref[...] = bmax
        xs_ref[...] = bsum

    @pl.when(nb > 0)
    def _acc():
        xm_ref[...] = jnp.maximum(xm_ref[...], bmax)
        xs_ref[...] = xs_ref[...] + bsum


def _mlp_body(xm_ref, xs_ref, l1_ref, g6_ref, b6_ref,
              l2_ref, l2b_ref, g7_ref, b7_ref, l3_ref, l3b_ref, out_ref,
              *, n_points):
    hcat = jnp.concatenate([xm_ref[:, 0, :], xs_ref[:, 0, :] * (1.0 / n_points)],
                           axis=1)
    h1 = lax.dot_general(hcat, l1_ref[...], (((1,), (1,)), ((), ())),
                         preferred_element_type=jnp.float32)
    h1 = _lrelu(h1 * (g6_ref[...][0] * _BN_RSQRT)[None, :] + b6_ref[...][0][None, :])
    h2 = lax.dot_general(h1, l2_ref[...], (((1,), (1,)), ((), ())),
                         preferred_element_type=jnp.float32) + l2b_ref[...][0][None, :]
    h2 = _lrelu(h2 * (g7_ref[...][0] * _BN_RSQRT)[None, :] + b7_ref[...][0][None, :])
    out_ref[...] = lax.dot_general(h2, l3_ref[...], (((1,), (1,)), ((), ())),
                                   preferred_element_type=jnp.float32) + l3b_ref[...][0][None, :]


def _head(xc, p):
    B, N, C5 = xc.shape
    BN = 512
    xm, xs = pl.pallas_call(
        _conv5_pool_body,
        grid=(B, N // BN),
        in_specs=[
            pl.BlockSpec((1, BN, C5), lambda b, n: (b, n, 0)),
            pl.BlockSpec((C5, 1024), lambda b, n: (0, 0)),
            pl.BlockSpec((1, 1024), lambda b, n: (0, 0)),
            pl.BlockSpec((1, 1024), lambda b, n: (0, 0)),
        ],
        out_specs=[
            pl.BlockSpec((1, 1, 1024), lambda b, n: (b, 0, 0)),
            pl.BlockSpec((1, 1, 1024), lambda b, n: (b, 0, 0)),
        ],
        out_shape=[
            jax.ShapeDtypeStruct((B, 1, 1024), jnp.float32),
            jax.ShapeDtypeStruct((B, 1, 1024), jnp.float32),
        ],
    )(xc, p['conv5_w'].T, p['bn5_g'][None, :], p['bn5_b'][None, :])
    return pl.pallas_call(
        functools.partial(_mlp_body, n_points=N),
        out_shape=jax.ShapeDtypeStruct((B, 40), jnp.float32),
    )(xm, xs, p['lin1_w'], p['bn6_g'][None, :], p['bn6_b'][None, :],
      p['lin2_w'], p['lin2_b'][None, :], p['bn7_g'][None, :],
      p['bn7_b'][None, :], p['lin3_w'], p['lin3_b'][None, :])


def kernel(x, params):
    p = params
    B, _, N = x.shape
    cur = jnp.transpose(x, (0, 2, 1))
    feats = []
    for i in range(4):
        C = cur.shape[2]
        idx, avg = _pd_topk(cur)
        k = _kcalc(avg)
        ks = k[0, 0]
        maskf = (jnp.arange(KMAX, dtype=jnp.int32)[None, :]
                 < ks).astype(jnp.float32)
        kf = ks.astype(jnp.float32)[None, None]

        w = p['conv%d_w' % (i + 1)]
        Co = w.shape[0]
        wa, wb = w[:, :C], w[:, C:]
        s = p['bn%d_g' % (i + 1)] * _BN_RSQRT
        wu = wa.T * s[None, :]
        wv = (wb - wa).T * s[None, :]
        bb = (p['bn%d_b' % (i + 1)])[None, :]
        U, V = _uv(cur, wu, wv, bb)

        idxf = (idx + (jnp.arange(B, dtype=jnp.int32) * N)[:, None, None]
                ).reshape(-1)
        Cop = max(Co, 128)
        Upad = U.reshape(B * N, Co)
        if Cop != Co:
            Upad = jnp.pad(Upad, ((0, 0), (0, Cop - Co)))
        E = _sc_gather(Upad, idxf, B * N * KMAX)
        E4 = E.reshape(B, N, KMAX, Cop)[..., :Co]

        ssum, smax = _edge_stats(E4, V, maskf)
        attc = _attc(ssum, smax, p['ca%d_fc1' % (i + 1)],
                     p['ca%d_fc2' % (i + 1)], kf, N)
        mp = _edge_map(E4, V, attc, maskf)
        mapp = jnp.pad(mp, ((0, 0), (0, 0), (2, 2), (2, 2)))
        sw_flat = p['sa%d_w' % (i + 1)].reshape(-1)
        cur = _edge_final(E4, V, attc, mapp, sw_flat, maskf)
        feats.append(cur)
    xc = jnp.concatenate(feats, axis=2)
    return _head(xc, p)
